# baseline (speedup 1.0000x reference)
import jax
import jax.numpy as jnp
from jax import lax
from jax.experimental import pallas as pl
from jax.experimental.pallas import tpu as pltpu
from jax.experimental.pallas import tpu_sc as plsc

VOCAB = 1000000
EMBED_DIM = 64
HALF = EMBED_DIM // 2
BATCH = 4096
HIST = 200

_INFO = plsc.get_sparse_core_info()
NC, NS, L = _INFO.num_cores, _INFO.num_subcores, _INFO.num_lanes
NW = NC * NS

B = BATCH * HIST
B_PER_W = B // NW
BATCH_PER_W = BATCH // NW
N_CHUNKS = BATCH_PER_W
NBUF = 4
LOOKAHEAD = 2


def _gather_body(x_hbm, tabL, tabR, out_hbm, idx_v, rowsL, rowsR,
                 g0, g1, g2, g3, o0, o1, o2, o3):
    gsems = (g0, g1, g2, g3)
    osems = (o0, o1, o2, o3)
    wid = lax.axis_index("s") * NC + lax.axis_index("c")
    base = wid * B_PER_W
    b_base = wid * BATCH_PER_W
    pltpu.sync_copy(x_hbm.at[pl.ds(base, B_PER_W)], idx_v)

    def gather_copies(c, b, construct_only=False):
        idx = idx_v.at[pl.ds(c * HIST, HIST)]
        mk = pltpu.make_async_copy if construct_only else pltpu.async_copy
        return (mk(tabL.at[idx], rowsL.at[b], gsems[b]),
                mk(tabR.at[idx], rowsR.at[b], gsems[b]))

    def fire_gather(c, b):
        gather_copies(c, b)

    def wait_gather(c, b):
        for cp in gather_copies(c, b, construct_only=True):
            cp.wait()

    def out_copies(c, b, construct_only=False):
        mk = pltpu.make_async_copy if construct_only else pltpu.async_copy
        return (mk(rowsL.at[b], out_hbm.at[b_base + c, :, pl.ds(0, HALF)],
                   osems[b]),
                mk(rowsR.at[b], out_hbm.at[b_base + c, :, pl.ds(HALF, HALF)],
                   osems[b]))

    def fire_out(c, b):
        out_copies(c, b)

    def wait_out(c, b):
        for cp in out_copies(c, b, construct_only=True):
            cp.wait()

    for c in range(LOOKAHEAD):
        fire_gather(c, c % NBUF)

    def round_body(r, carry):
        for b in range(NBUF):
            c = r * NBUF + b
            c2 = c + LOOKAHEAD
            b2 = (b + LOOKAHEAD) % NBUF

            @pl.when(c2 < N_CHUNKS)
            def _():
                @pl.when(c2 >= NBUF)
                def _():
                    wait_out(c2 - NBUF, b2)
                fire_gather(c2, b2)

            wait_gather(c, b)
            fire_out(c, b)
        return carry

    lax.fori_loop(0, N_CHUNKS // NBUF, round_body, 0)

    for k in range(NBUF):
        c = N_CHUNKS - NBUF + k
        wait_out(c, c % NBUF)


def kernel(x, table):
    x1 = x.reshape(B).astype(jnp.int32)
    tabL = table[:, :HALF]
    tabR = table[:, HALF:]
    mesh = plsc.VectorSubcoreMesh(core_axis_name="c", subcore_axis_name="s")
    padded = pl.kernel(
        _gather_body,
        mesh=mesh,
        out_type=jax.ShapeDtypeStruct((BATCH, HIST, 2 * EMBED_DIM), jnp.float32),
        scratch_types=[
            pltpu.VMEM((B_PER_W,), jnp.int32),
            pltpu.VMEM((NBUF, HIST, HALF), jnp.float32),
            pltpu.VMEM((NBUF, HIST, HALF), jnp.float32),
        ] + [pltpu.SemaphoreType.DMA] * (2 * NBUF),
        compiler_params=pltpu.CompilerParams(use_tc_tiling_on_sc=False),
    )(x1, tabL, tabR)
    return padded[:, :, :EMBED_DIM]


# speedup vs baseline: 1.8430x; 1.8430x over previous
import jax
import jax.numpy as jnp
from jax import lax
from jax.experimental import pallas as pl
from jax.experimental.pallas import tpu as pltpu
from jax.experimental.pallas import tpu_sc as plsc

VOCAB = 1000000
EMBED_DIM = 64
BATCH = 4096
HIST = 200

_INFO = plsc.get_sparse_core_info()
NC, NS, L = _INFO.num_cores, _INFO.num_subcores, _INFO.num_lanes
NW = NC * NS

B = BATCH * HIST
B_PER_W = B // NW
BATCH_PER_W = BATCH // NW
N_CHUNKS = BATCH_PER_W
NBUF = 4
LOOKAHEAD = 2


def _gather_body(x_hbm, table_hbm, out_hbm, idx_v, rows_v,
                 g0, g1, g2, g3, o0, o1, o2, o3):
    gsems = (g0, g1, g2, g3)
    osems = (o0, o1, o2, o3)
    wid = lax.axis_index("s") * NC + lax.axis_index("c")
    base = wid * B_PER_W
    b_base = wid * BATCH_PER_W
    pltpu.sync_copy(x_hbm.at[pl.ds(base, B_PER_W)], idx_v)

    def fire_gather(c, b):
        pltpu.async_copy(
            table_hbm.at[idx_v.at[pl.ds(c * HIST, HIST)]],
            rows_v.at[b], gsems[b])

    def wait_gather(c, b):
        pltpu.make_async_copy(
            table_hbm.at[idx_v.at[pl.ds(c * HIST, HIST)]],
            rows_v.at[b], gsems[b]).wait()

    def fire_out(c, b):
        pltpu.async_copy(
            rows_v.at[b],
            out_hbm.at[b_base + c, :, pl.ds(0, EMBED_DIM)], osems[b])

    def wait_out(c, b):
        pltpu.make_async_copy(
            rows_v.at[b],
            out_hbm.at[b_base + c, :, pl.ds(0, EMBED_DIM)], osems[b]).wait()

    for c in range(LOOKAHEAD):
        fire_gather(c, c % NBUF)

    def round_body(r, carry):
        for b in range(NBUF):
            c = r * NBUF + b
            c2 = c + LOOKAHEAD
            b2 = (b + LOOKAHEAD) % NBUF

            @pl.when(c2 < N_CHUNKS)
            def _():
                @pl.when(c2 >= NBUF)
                def _():
                    wait_out(c2 - NBUF, b2)
                fire_gather(c2, b2)

            wait_gather(c, b)
            fire_out(c, b)
        return carry

    lax.fori_loop(0, N_CHUNKS // NBUF, round_body, 0)

    for k in range(NBUF):
        c = N_CHUNKS - NBUF + k
        wait_out(c, c % NBUF)


def kernel(x, table):
    x1 = x.reshape(B).astype(jnp.int32)
    mesh = plsc.VectorSubcoreMesh(core_axis_name="c", subcore_axis_name="s")
    padded = pl.kernel(
        _gather_body,
        mesh=mesh,
        out_type=jax.ShapeDtypeStruct((BATCH, HIST, 2 * EMBED_DIM), jnp.float32),
        scratch_types=[
            pltpu.VMEM((B_PER_W,), jnp.int32),
            pltpu.VMEM((NBUF, HIST, EMBED_DIM), jnp.float32),
        ] + [pltpu.SemaphoreType.DMA] * (2 * NBUF),
        compiler_params=pltpu.CompilerParams(use_tc_tiling_on_sc=False),
    )(x1, table)
    return padded[:, :, :EMBED_DIM]
